# baseline (speedup 1.0000x reference)
import functools

import jax
import jax.numpy as jnp
from jax import lax
from jax.experimental import pallas as pl
from jax.experimental.pallas import tpu as pltpu
from jax.experimental.pallas import tpu_sc as plsc

N, H, C, D = 16384, 512, 1000, 512
BN = 512
_EPS = 1e-8

_NC, _NS = 2, 16
_NW = _NC * _NS
_BPW = N // _NW
_CH = 64
_NCH = _BPW // _CH


def _prep_body(anchors_ref, w1_ref, anorm_ref, ap_ref):
    a = anchors_ref[...]
    norm = jnp.sqrt(jnp.sum(a * a, axis=1, keepdims=True))
    anorm_ref[...] = (a / jnp.maximum(norm, _EPS)).astype(jnp.bfloat16)
    ap_ref[...] = lax.dot_general(
        a, w1_ref[...], (((1,), (1,)), ((), ())),
        preferred_element_type=jnp.float32)


def _sim_body(f_ref, anorm_ref, idx_ref):
    sim = lax.dot_general(
        f_ref[...].astype(jnp.bfloat16), anorm_ref[...],
        (((1,), (1,)), ((), ())),
        preferred_element_type=jnp.float32)
    mx = jnp.max(sim, axis=-1, keepdims=True)
    onehot = jnp.where(sim == mx, 1.0, 0.0).astype(jnp.float32)
    iota = lax.broadcasted_iota(jnp.int32, (C, 128), 0).astype(jnp.float32)
    idxf = lax.dot_general(
        onehot, iota, (((1,), (0,)), ((), ())),
        preferred_element_type=jnp.float32)
    idx_ref[...] = jnp.minimum(idxf[:, :1], float(C - 1)).astype(jnp.int32)


def _proj_body(f_ref, g_ref, w2_ref, out_ref):
    out_ref[...] = g_ref[...] + lax.dot_general(
        f_ref[...], w2_ref[...], (((1,), (1,)), ((), ())),
        preferred_element_type=jnp.float32)


def _gather_body(ap_hbm, idx_hbm, out_hbm, idx_v, rows_v, sem0, sem1):
    wid = lax.axis_index("s") * _NC + lax.axis_index("c")
    pltpu.sync_copy(idx_hbm.at[pl.ds(wid * _NCH, _NCH)], idx_v)
    base = wid * _BPW
    sems = (sem0, sem1)
    handles = [None] * _NCH
    handles[0] = pltpu.async_copy(ap_hbm.at[idx_v.at[0]], rows_v.at[0], sem0)
    for j in range(_NCH):
        if j + 1 < _NCH:
            handles[j + 1] = pltpu.async_copy(
                ap_hbm.at[idx_v.at[j + 1]], rows_v.at[(j + 1) % 2],
                sems[(j + 1) % 2])
        handles[j].wait()
        pltpu.sync_copy(rows_v.at[j % 2],
                        out_hbm.at[pl.ds(base + j * _CH, _CH)])


@functools.cache
def _gather_call():
    return functools.partial(
        pl.kernel,
        mesh=plsc.VectorSubcoreMesh(
            core_axis_name="c", subcore_axis_name="s", num_cores=_NC),
        out_type=jax.ShapeDtypeStruct((N, D), jnp.float32),
        scratch_types=[
            pltpu.VMEM((_NCH, _CH), jnp.int32),
            pltpu.VMEM((2, _CH, D), jnp.float32),
            pltpu.SemaphoreType.DMA,
            pltpu.SemaphoreType.DMA,
        ],
    )(_gather_body)


def kernel(features, class_anchors, W_proj):
    anorm, ap = pl.pallas_call(
        _prep_body,
        grid=(1,),
        in_specs=[
            pl.BlockSpec((C, H), lambda i: (0, 0)),
            pl.BlockSpec((D, H), lambda i: (0, 0)),
        ],
        out_specs=(pl.BlockSpec((C, H), lambda i: (0, 0)),
                   pl.BlockSpec((C, D), lambda i: (0, 0))),
        out_shape=(jax.ShapeDtypeStruct((C, H), jnp.bfloat16),
                   jax.ShapeDtypeStruct((C, D), jnp.float32)),
    )(class_anchors, W_proj)

    idx = pl.pallas_call(
        _sim_body,
        grid=(N // BN,),
        in_specs=[
            pl.BlockSpec((BN, H), lambda i: (i, 0)),
            pl.BlockSpec((C, H), lambda i: (0, 0)),
        ],
        out_specs=pl.BlockSpec((BN, 1), lambda i: (i, 0)),
        out_shape=jax.ShapeDtypeStruct((N, 1), jnp.int32),
    )(features, anorm)

    g = _gather_call()(ap, idx.reshape(_NW * _NCH, _CH))

    out = pl.pallas_call(
        _proj_body,
        grid=(N // BN,),
        in_specs=[
            pl.BlockSpec((BN, H), lambda i: (i, 0)),
            pl.BlockSpec((BN, D), lambda i: (i, 0)),
            pl.BlockSpec((D, H), lambda i: (0, 1)),
        ],
        out_specs=pl.BlockSpec((BN, D), lambda i: (i, 0)),
        out_shape=jax.ShapeDtypeStruct((N, D), jnp.float32),
    )(features, g, W_proj)
    return out


# speedup vs baseline: 1.0767x; 1.0767x over previous
import functools

import jax
import jax.numpy as jnp
from jax import lax
from jax.experimental import pallas as pl
from jax.experimental.pallas import tpu as pltpu
from jax.experimental.pallas import tpu_sc as plsc

N, H, C, D = 16384, 512, 1000, 512
BN = 512
_EPS = 1e-8

_NC, _NS = 2, 16
_NW = _NC * _NS
_BPW = N // _NW
_CH = 64
_NCH = _BPW // _CH


def _prep_body(anchors_ref, w1_ref, anorm_ref, ap_ref):
    a = anchors_ref[...]
    norm = jnp.sqrt(jnp.sum(a * a, axis=1, keepdims=True))
    anorm_ref[...] = (a / jnp.maximum(norm, _EPS)).astype(jnp.bfloat16)
    ap_ref[...] = lax.dot_general(
        a, w1_ref[...], (((1,), (1,)), ((), ())),
        preferred_element_type=jnp.float32)


def _sim_body(f_ref, anorm_ref, idx_ref):
    sim = lax.dot_general(
        f_ref[...].astype(jnp.bfloat16), anorm_ref[...],
        (((1,), (1,)), ((), ())),
        preferred_element_type=jnp.float32)
    mx = jnp.max(sim, axis=-1, keepdims=True)
    onehot = jnp.where(sim == mx, 1.0, 0.0).astype(jnp.float32)
    iota = lax.broadcasted_iota(jnp.int32, (C, 128), 0).astype(jnp.float32)
    idxf = lax.dot_general(
        onehot, iota, (((1,), (0,)), ((), ())),
        preferred_element_type=jnp.float32)
    idx_ref[...] = jnp.minimum(idxf[:, :1] + 0.5, float(C - 1)).astype(jnp.int32)


def _proj_body(f_ref, g_ref, w2_ref, out_ref):
    out_ref[...] = g_ref[...] + lax.dot_general(
        f_ref[...], w2_ref[...], (((1,), (1,)), ((), ())),
        preferred_element_type=jnp.float32)


def _gather_body(ap_hbm, idx_hbm, out_hbm, idx_v, rows_v, sem0, sem1):
    wid = lax.axis_index("s") * _NC + lax.axis_index("c")
    pltpu.sync_copy(idx_hbm.at[pl.ds(wid * _NCH, _NCH)], idx_v)
    base = wid * _BPW
    sems = (sem0, sem1)
    handles = [None] * _NCH
    handles[0] = pltpu.async_copy(ap_hbm.at[idx_v.at[0]], rows_v.at[0], sem0)
    for j in range(_NCH):
        if j + 1 < _NCH:
            handles[j + 1] = pltpu.async_copy(
                ap_hbm.at[idx_v.at[j + 1]], rows_v.at[(j + 1) % 2],
                sems[(j + 1) % 2])
        handles[j].wait()
        pltpu.sync_copy(rows_v.at[j % 2],
                        out_hbm.at[pl.ds(base + j * _CH, _CH)])


@functools.cache
def _gather_call():
    return functools.partial(
        pl.kernel,
        mesh=plsc.VectorSubcoreMesh(
            core_axis_name="c", subcore_axis_name="s", num_cores=_NC),
        out_type=jax.ShapeDtypeStruct((N, D), jnp.float32),
        scratch_types=[
            pltpu.VMEM((_NCH, _CH), jnp.int32),
            pltpu.VMEM((2, _CH, D), jnp.float32),
            pltpu.SemaphoreType.DMA,
            pltpu.SemaphoreType.DMA,
        ],
    )(_gather_body)


def kernel(features, class_anchors, W_proj):
    anorm, ap = pl.pallas_call(
        _prep_body,
        grid=(1,),
        in_specs=[
            pl.BlockSpec((C, H), lambda i: (0, 0)),
            pl.BlockSpec((D, H), lambda i: (0, 0)),
        ],
        out_specs=(pl.BlockSpec((C, H), lambda i: (0, 0)),
                   pl.BlockSpec((C, D), lambda i: (0, 0))),
        out_shape=(jax.ShapeDtypeStruct((C, H), jnp.bfloat16),
                   jax.ShapeDtypeStruct((C, D), jnp.float32)),
    )(class_anchors, W_proj)

    idx = pl.pallas_call(
        _sim_body,
        grid=(N // BN,),
        in_specs=[
            pl.BlockSpec((BN, H), lambda i: (i, 0)),
            pl.BlockSpec((C, H), lambda i: (0, 0)),
        ],
        out_specs=pl.BlockSpec((BN, 1), lambda i: (i, 0)),
        out_shape=jax.ShapeDtypeStruct((N, 1), jnp.int32),
    )(features, anorm)

    g = _gather_call()(ap, idx.reshape(_NW * _NCH, _CH))

    out = pl.pallas_call(
        _proj_body,
        grid=(N // BN,),
        in_specs=[
            pl.BlockSpec((BN, H), lambda i: (i, 0)),
            pl.BlockSpec((BN, D), lambda i: (i, 0)),
            pl.BlockSpec((D, H), lambda i: (0, 1)),
        ],
        out_specs=pl.BlockSpec((BN, D), lambda i: (i, 0)),
        out_shape=jax.ShapeDtypeStruct((N, D), jnp.float32),
    )(features, g, W_proj)
    return out


# speedup vs baseline: 1.3059x; 1.2129x over previous
import functools

import jax
import jax.numpy as jnp
from jax import lax
from jax.experimental import pallas as pl
from jax.experimental.pallas import tpu as pltpu
from jax.experimental.pallas import tpu_sc as plsc

N, H, C, D = 16384, 512, 1000, 512
BN = 512
_EPS = 1e-8

_NC, _NS = 2, 16
_NW = _NC * _NS
_BPW = N // _NW
_CH = 128
_NCH = _BPW // _CH
_DP = D // 2


def _prep_body(anchors_ref, w1_ref, anorm_ref, ap_ref):
    a = anchors_ref[...]
    norm = jnp.sqrt(jnp.sum(a * a, axis=1, keepdims=True))
    anorm_ref[...] = (a / jnp.maximum(norm, _EPS)).astype(jnp.bfloat16)
    ap_ref[...] = lax.dot_general(
        a, w1_ref[...], (((1,), (1,)), ((), ())),
        preferred_element_type=jnp.float32)


def _sim_body(f_ref, anorm_ref, idx_ref):
    sim = lax.dot_general(
        f_ref[...].astype(jnp.bfloat16), anorm_ref[...],
        (((1,), (1,)), ((), ())),
        preferred_element_type=jnp.float32)
    idx_ref[...] = jnp.argmax(sim, axis=-1).astype(jnp.int32)[None, None, :]


def _proj_body(f_ref, g_ref, w2_ref, out_ref):
    gw = g_ref[...]
    lo = lax.bitcast_convert_type(gw << 16, jnp.float32)
    hi = lax.bitcast_convert_type(gw & jnp.int32(-65536), jnp.float32)
    g = jnp.concatenate([lo, hi], axis=1)
    out_ref[...] = g + lax.dot_general(
        f_ref[...], w2_ref[...], (((1,), (1,)), ((), ())),
        preferred_element_type=jnp.float32)


def _gather_body(ap_hbm, idx_hbm, out_hbm, idx_v, rows_v, sem0, sem1):
    wid = lax.axis_index("s") * _NC + lax.axis_index("c")
    pltpu.sync_copy(idx_hbm.at[pl.ds(wid * _NCH, _NCH)], idx_v)
    base = wid * _BPW
    sems = (sem0, sem1)
    handles = [None] * _NCH
    handles[0] = pltpu.async_copy(ap_hbm.at[idx_v.at[0]], rows_v.at[0], sem0)
    for j in range(_NCH):
        if j + 1 < _NCH:
            handles[j + 1] = pltpu.async_copy(
                ap_hbm.at[idx_v.at[j + 1]], rows_v.at[(j + 1) % 2],
                sems[(j + 1) % 2])
        handles[j].wait()
        pltpu.sync_copy(rows_v.at[j % 2],
                        out_hbm.at[pl.ds(base + j * _CH, _CH)])


@functools.cache
def _gather_call():
    return functools.partial(
        pl.kernel,
        mesh=plsc.VectorSubcoreMesh(
            core_axis_name="c", subcore_axis_name="s", num_cores=_NC),
        out_type=jax.ShapeDtypeStruct((N, _DP), jnp.int32),
        scratch_types=[
            pltpu.VMEM((_NCH, _CH), jnp.int32),
            pltpu.VMEM((2, _CH, _DP), jnp.int32),
            pltpu.SemaphoreType.DMA,
            pltpu.SemaphoreType.DMA,
        ],
    )(_gather_body)


def kernel(features, class_anchors, W_proj):
    anorm, ap = pl.pallas_call(
        _prep_body,
        grid=(1,),
        in_specs=[
            pl.BlockSpec((C, H), lambda i: (0, 0)),
            pl.BlockSpec((D, H), lambda i: (0, 0)),
        ],
        out_specs=(pl.BlockSpec((C, H), lambda i: (0, 0)),
                   pl.BlockSpec((C, D), lambda i: (0, 0))),
        out_shape=(jax.ShapeDtypeStruct((C, H), jnp.bfloat16),
                   jax.ShapeDtypeStruct((C, D), jnp.float32)),
    )(class_anchors, W_proj)

    ap_b = ap.astype(jnp.bfloat16)
    lo_u = lax.bitcast_convert_type(ap_b[:, :_DP], jnp.uint16).astype(jnp.uint32)
    hi_u = lax.bitcast_convert_type(ap_b[:, _DP:], jnp.uint16).astype(jnp.uint32)
    ap_packed = lax.bitcast_convert_type(lo_u | (hi_u << 16), jnp.int32)

    idx = pl.pallas_call(
        _sim_body,
        grid=(N // BN,),
        in_specs=[
            pl.BlockSpec((BN, H), lambda i: (i, 0)),
            pl.BlockSpec((C, H), lambda i: (0, 0)),
        ],
        out_specs=pl.BlockSpec((1, 1, BN), lambda i: (i, 0, 0)),
        out_shape=jax.ShapeDtypeStruct((N // BN, 1, BN), jnp.int32),
    )(features, anorm)

    g = _gather_call()(ap_packed, idx.reshape(_NW * _NCH, _CH))

    out = pl.pallas_call(
        _proj_body,
        grid=(N // BN,),
        in_specs=[
            pl.BlockSpec((BN, H), lambda i: (i, 0)),
            pl.BlockSpec((BN, _DP), lambda i: (i, 0)),
            pl.BlockSpec((D, H), lambda i: (0, 1)),
        ],
        out_specs=pl.BlockSpec((BN, D), lambda i: (i, 0)),
        out_shape=jax.ShapeDtypeStruct((N, D), jnp.float32),
    )(features, g, W_proj)
    return out


# speedup vs baseline: 1.5068x; 1.1538x over previous
import functools

import jax
import jax.numpy as jnp
from jax import lax
from jax.experimental import pallas as pl
from jax.experimental.pallas import tpu as pltpu
from jax.experimental.pallas import tpu_sc as plsc

N, H, C, D = 16384, 512, 1000, 512
BN = 1024
_EPS = 1e-8

_NC, _NS = 2, 16
_NW = _NC * _NS
_BPW = N // _NW
_CH = 128
_NCH = _BPW // _CH
_DP = D // 2


def _prep_body(anchors_ref, w1_ref, anorm_ref, ap_ref):
    a = anchors_ref[...]
    norm = jnp.sqrt(jnp.sum(a * a, axis=1, keepdims=True))
    anorm_ref[...] = (a / jnp.maximum(norm, _EPS)).astype(jnp.bfloat16)
    ap = lax.dot_general(
        a, w1_ref[...], (((1,), (1,)), ((), ())),
        preferred_element_type=jnp.float32)
    bits = lax.bitcast_convert_type(ap, jnp.uint32) + jnp.uint32(0x8000)
    lo = bits[:, :_DP] >> 16
    hi = bits[:, _DP:] & jnp.uint32(0xFFFF0000)
    ap_ref[...] = lax.bitcast_convert_type(lo | hi, jnp.int32)


def _sim_body(f_ref, anorm_ref, idx_ref):
    sim = lax.dot_general(
        f_ref[...].astype(jnp.bfloat16), anorm_ref[...],
        (((1,), (1,)), ((), ())),
        preferred_element_type=jnp.float32)
    idx_ref[...] = jnp.argmax(sim, axis=-1).astype(jnp.int32)[None, None, :]


def _proj_body(f_ref, g_ref, w2_ref, out_ref):
    gw = g_ref[...]
    lo = lax.bitcast_convert_type(gw << 16, jnp.float32)
    hi = lax.bitcast_convert_type(gw & jnp.int32(-65536), jnp.float32)
    g = jnp.concatenate([lo, hi], axis=1)
    out_ref[...] = g + lax.dot_general(
        f_ref[...], w2_ref[...], (((1,), (1,)), ((), ())),
        preferred_element_type=jnp.float32)


def _gather_body(ap_hbm, idx_hbm, out_hbm, idx_v, rows_v, sem0, sem1):
    wid = lax.axis_index("s") * _NC + lax.axis_index("c")
    pltpu.sync_copy(idx_hbm.at[pl.ds(wid * _NCH, _NCH)], idx_v)
    base = wid * _BPW
    sems = (sem0, sem1)
    handles = [None] * _NCH
    handles[0] = pltpu.async_copy(ap_hbm.at[idx_v.at[0]], rows_v.at[0], sem0)
    for j in range(_NCH):
        if j + 1 < _NCH:
            handles[j + 1] = pltpu.async_copy(
                ap_hbm.at[idx_v.at[j + 1]], rows_v.at[(j + 1) % 2],
                sems[(j + 1) % 2])
        handles[j].wait()
        pltpu.sync_copy(rows_v.at[j % 2],
                        out_hbm.at[pl.ds(base + j * _CH, _CH)])


@functools.cache
def _gather_call():
    return functools.partial(
        pl.kernel,
        mesh=plsc.VectorSubcoreMesh(
            core_axis_name="c", subcore_axis_name="s", num_cores=_NC),
        out_type=jax.ShapeDtypeStruct((N, _DP), jnp.int32),
        scratch_types=[
            pltpu.VMEM((_NCH, _CH), jnp.int32),
            pltpu.VMEM((2, _CH, _DP), jnp.int32),
            pltpu.SemaphoreType.DMA,
            pltpu.SemaphoreType.DMA,
        ],
    )(_gather_body)


def kernel(features, class_anchors, W_proj):
    anorm, ap = pl.pallas_call(
        _prep_body,
        grid=(1,),
        in_specs=[
            pl.BlockSpec((C, H), lambda i: (0, 0)),
            pl.BlockSpec((D, H), lambda i: (0, 0)),
        ],
        out_specs=(pl.BlockSpec((C, H), lambda i: (0, 0)),
                   pl.BlockSpec((C, _DP), lambda i: (0, 0))),
        out_shape=(jax.ShapeDtypeStruct((C, H), jnp.bfloat16),
                   jax.ShapeDtypeStruct((C, _DP), jnp.int32)),
    )(class_anchors, W_proj)
    ap_packed = ap

    idx = pl.pallas_call(
        _sim_body,
        grid=(N // BN,),
        in_specs=[
            pl.BlockSpec((BN, H), lambda i: (i, 0)),
            pl.BlockSpec((C, H), lambda i: (0, 0)),
        ],
        out_specs=pl.BlockSpec((1, 1, BN), lambda i: (i, 0, 0)),
        out_shape=jax.ShapeDtypeStruct((N // BN, 1, BN), jnp.int32),
    )(features, anorm)

    g = _gather_call()(ap_packed, idx.reshape(_NW * _NCH, _CH))

    out = pl.pallas_call(
        _proj_body,
        grid=(N // BN,),
        in_specs=[
            pl.BlockSpec((BN, H), lambda i: (i, 0)),
            pl.BlockSpec((BN, _DP), lambda i: (i, 0)),
            pl.BlockSpec((D, H), lambda i: (0, 1)),
        ],
        out_specs=pl.BlockSpec((BN, D), lambda i: (i, 0)),
        out_shape=jax.ShapeDtypeStruct((N, D), jnp.float32),
    )(features, g, W_proj)
    return out


# speedup vs baseline: 1.5808x; 1.0491x over previous
import functools

import jax
import jax.numpy as jnp
from jax import lax
from jax.experimental import pallas as pl
from jax.experimental.pallas import tpu as pltpu
from jax.experimental.pallas import tpu_sc as plsc

N, H, C, D = 16384, 512, 1000, 512
BN = 1024
_EPS = 1e-8

_NC, _NS = 2, 16
_NW = _NC * _NS
_NH = N // 2
_BPW = _NH // _NW
_CH = 128
_NCH = _BPW // _CH
_DP = D // 2
_GB = _NH // BN


def _prep_body(anchors_ref, w1_ref, anorm_ref, ap_ref):
    a = anchors_ref[...]
    norm = jnp.sqrt(jnp.sum(a * a, axis=1, keepdims=True))
    anorm_ref[...] = (a / jnp.maximum(norm, _EPS)).astype(jnp.bfloat16)
    ap = lax.dot_general(
        a, w1_ref[...], (((1,), (1,)), ((), ())),
        preferred_element_type=jnp.float32)
    bits = lax.bitcast_convert_type(ap, jnp.uint32) + jnp.uint32(0x8000)
    lo = bits[:, :_DP] >> 16
    hi = bits[:, _DP:] & jnp.uint32(0xFFFF0000)
    ap_ref[...] = lax.bitcast_convert_type(lo | hi, jnp.int32)


def _sim_body(f_ref, anorm_ref, idx_ref):
    sim = lax.dot_general(
        f_ref[...].astype(jnp.bfloat16), anorm_ref[...],
        (((1,), (1,)), ((), ())),
        preferred_element_type=jnp.float32)
    idx_ref[...] = jnp.argmax(sim, axis=-1).astype(jnp.int32)[None, None, :]


def _unpack_add(f_ref, g_ref, w2_ref, out_ref):
    gw = g_ref[...]
    lo = lax.bitcast_convert_type(gw << 16, jnp.float32)
    hi = lax.bitcast_convert_type(gw & jnp.int32(-65536), jnp.float32)
    g = jnp.concatenate([lo, hi], axis=1)
    out_ref[...] = g + lax.dot_general(
        f_ref[...], w2_ref[...], (((1,), (1,)), ((), ())),
        preferred_element_type=jnp.float32)


def _proj_body_a(f_ref, g_ref, w2_ref, out_ref):
    _unpack_add(f_ref, g_ref, w2_ref, out_ref)


def _proj_body_b(prev_ref, f_ref, g_ref, w2_ref, out_ref):
    del prev_ref
    _unpack_add(f_ref, g_ref, w2_ref, out_ref)


def _gather_body(ap_hbm, idx_hbm, out_hbm, idx_v, rows_v, sem0, sem1):
    wid = lax.axis_index("s") * _NC + lax.axis_index("c")
    pltpu.sync_copy(idx_hbm.at[pl.ds(wid * _NCH, _NCH)], idx_v)
    base = wid * _BPW
    sems = (sem0, sem1)
    handles = [None] * _NCH
    handles[0] = pltpu.async_copy(ap_hbm.at[idx_v.at[0]], rows_v.at[0], sem0)
    for j in range(_NCH):
        if j + 1 < _NCH:
            handles[j + 1] = pltpu.async_copy(
                ap_hbm.at[idx_v.at[j + 1]], rows_v.at[(j + 1) % 2],
                sems[(j + 1) % 2])
        handles[j].wait()
        pltpu.sync_copy(rows_v.at[j % 2],
                        out_hbm.at[pl.ds(base + j * _CH, _CH)])


@functools.cache
def _gather_call():
    return functools.partial(
        pl.kernel,
        mesh=plsc.VectorSubcoreMesh(
            core_axis_name="c", subcore_axis_name="s", num_cores=_NC),
        out_type=jax.ShapeDtypeStruct((_NH, _DP), jnp.int32),
        scratch_types=[
            pltpu.VMEM((_NCH, _CH), jnp.int32),
            pltpu.VMEM((2, _CH, _DP), jnp.int32),
            pltpu.SemaphoreType.DMA,
            pltpu.SemaphoreType.DMA,
        ],
    )(_gather_body)


def _sim_half(features, anorm, block0):
    return pl.pallas_call(
        _sim_body,
        grid=(_GB,),
        in_specs=[
            pl.BlockSpec((BN, H), lambda i: (i + block0, 0)),
            pl.BlockSpec((C, H), lambda i: (0, 0)),
        ],
        out_specs=pl.BlockSpec((1, 1, BN), lambda i: (i, 0, 0)),
        out_shape=jax.ShapeDtypeStruct((_GB, 1, BN), jnp.int32),
    )(features, anorm)


def kernel(features, class_anchors, W_proj):
    anorm, ap = pl.pallas_call(
        _prep_body,
        grid=(1,),
        in_specs=[
            pl.BlockSpec((C, H), lambda i: (0, 0)),
            pl.BlockSpec((D, H), lambda i: (0, 0)),
        ],
        out_specs=(pl.BlockSpec((C, H), lambda i: (0, 0)),
                   pl.BlockSpec((C, _DP), lambda i: (0, 0))),
        out_shape=(jax.ShapeDtypeStruct((C, H), jnp.bfloat16),
                   jax.ShapeDtypeStruct((C, _DP), jnp.int32)),
    )(class_anchors, W_proj)

    idx0 = _sim_half(features, anorm, 0)
    idx1 = _sim_half(features, anorm, _GB)

    g0 = _gather_call()(ap, idx0.reshape(_NW * _NCH, _CH))
    g1 = _gather_call()(ap, idx1.reshape(_NW * _NCH, _CH))

    out_a = pl.pallas_call(
        _proj_body_a,
        grid=(_GB,),
        in_specs=[
            pl.BlockSpec((BN, H), lambda i: (i, 0)),
            pl.BlockSpec((BN, _DP), lambda i: (i, 0)),
            pl.BlockSpec((D, H), lambda i: (0, 1)),
        ],
        out_specs=pl.BlockSpec((BN, D), lambda i: (i, 0)),
        out_shape=jax.ShapeDtypeStruct((N, D), jnp.float32),
    )(features, g0, W_proj)

    out = pl.pallas_call(
        _proj_body_b,
        grid=(_GB,),
        in_specs=[
            pl.BlockSpec((8, 128), lambda i: (0, 0)),
            pl.BlockSpec((BN, H), lambda i: (i + _GB, 0)),
            pl.BlockSpec((BN, _DP), lambda i: (i, 0)),
            pl.BlockSpec((D, H), lambda i: (0, 1)),
        ],
        out_specs=pl.BlockSpec((BN, D), lambda i: (i + _GB, 0)),
        out_shape=jax.ShapeDtypeStruct((N, D), jnp.float32),
        input_output_aliases={0: 0},
    )(out_a, features, g1, W_proj)
    return out


# speedup vs baseline: 1.5823x; 1.0010x over previous
import functools

import jax
import jax.numpy as jnp
from jax import lax
from jax.experimental import pallas as pl
from jax.experimental.pallas import tpu as pltpu
from jax.experimental.pallas import tpu_sc as plsc

N, H, C, D = 16384, 512, 1000, 512
BN = 2048
_EPS = 1e-8

_NC, _NS = 2, 16
_NW = _NC * _NS
_NH = N // 2
_BPW = _NH // _NW
_CH = 128
_NCH = _BPW // _CH
_DP = D // 2
_GB = _NH // BN


def _prep_body(anchors_ref, w1_ref, anorm_ref, ap_ref):
    a = anchors_ref[...]
    norm = jnp.sqrt(jnp.sum(a * a, axis=1, keepdims=True))
    anorm_ref[...] = (a / jnp.maximum(norm, _EPS)).astype(jnp.bfloat16)
    ap = lax.dot_general(
        a, w1_ref[...], (((1,), (1,)), ((), ())),
        preferred_element_type=jnp.float32)
    bits = lax.bitcast_convert_type(ap, jnp.uint32) + jnp.uint32(0x8000)
    lo = bits[:, :_DP] >> 16
    hi = bits[:, _DP:] & jnp.uint32(0xFFFF0000)
    ap_ref[...] = lax.bitcast_convert_type(lo | hi, jnp.int32)


def _sim_body(f_ref, anorm_ref, idx_ref):
    sim = lax.dot_general(
        f_ref[...].astype(jnp.bfloat16), anorm_ref[...],
        (((1,), (1,)), ((), ())),
        preferred_element_type=jnp.float32)
    idx_ref[...] = jnp.argmax(sim, axis=-1).astype(jnp.int32)[None, None, :]


def _unpack_add(f_ref, g_ref, w2_ref, out_ref):
    gw = g_ref[...]
    lo = lax.bitcast_convert_type(gw << 16, jnp.float32)
    hi = lax.bitcast_convert_type(gw & jnp.int32(-65536), jnp.float32)
    g = jnp.concatenate([lo, hi], axis=1)
    out_ref[...] = g + lax.dot_general(
        f_ref[...], w2_ref[...], (((1,), (1,)), ((), ())),
        preferred_element_type=jnp.float32)


def _proj_body_a(f_ref, g_ref, w2_ref, out_ref):
    _unpack_add(f_ref, g_ref, w2_ref, out_ref)


def _proj_body_b(prev_ref, f_ref, g_ref, w2_ref, out_ref):
    del prev_ref
    _unpack_add(f_ref, g_ref, w2_ref, out_ref)


def _gather_body(ap_hbm, idx_hbm, out_hbm, idx_v, rows_v, sem0, sem1):
    wid = lax.axis_index("s") * _NC + lax.axis_index("c")
    pltpu.sync_copy(idx_hbm.at[pl.ds(wid * _NCH, _NCH)], idx_v)
    base = wid * _BPW
    sems = (sem0, sem1)
    handles = [None] * _NCH
    handles[0] = pltpu.async_copy(ap_hbm.at[idx_v.at[0]], rows_v.at[0], sem0)
    for j in range(_NCH):
        if j + 1 < _NCH:
            handles[j + 1] = pltpu.async_copy(
                ap_hbm.at[idx_v.at[j + 1]], rows_v.at[(j + 1) % 2],
                sems[(j + 1) % 2])
        handles[j].wait()
        pltpu.sync_copy(rows_v.at[j % 2],
                        out_hbm.at[pl.ds(base + j * _CH, _CH)])


@functools.cache
def _gather_call():
    return functools.partial(
        pl.kernel,
        mesh=plsc.VectorSubcoreMesh(
            core_axis_name="c", subcore_axis_name="s", num_cores=_NC),
        out_type=jax.ShapeDtypeStruct((_NH, _DP), jnp.int32),
        scratch_types=[
            pltpu.VMEM((_NCH, _CH), jnp.int32),
            pltpu.VMEM((2, _CH, _DP), jnp.int32),
            pltpu.SemaphoreType.DMA,
            pltpu.SemaphoreType.DMA,
        ],
    )(_gather_body)


def _sim_half(features, anorm, block0):
    return pl.pallas_call(
        _sim_body,
        grid=(_GB,),
        in_specs=[
            pl.BlockSpec((BN, H), lambda i: (i + block0, 0)),
            pl.BlockSpec((C, H), lambda i: (0, 0)),
        ],
        out_specs=pl.BlockSpec((1, 1, BN), lambda i: (i, 0, 0)),
        out_shape=jax.ShapeDtypeStruct((_GB, 1, BN), jnp.int32),
    )(features, anorm)


def kernel(features, class_anchors, W_proj):
    anorm, ap = pl.pallas_call(
        _prep_body,
        grid=(1,),
        in_specs=[
            pl.BlockSpec((C, H), lambda i: (0, 0)),
            pl.BlockSpec((D, H), lambda i: (0, 0)),
        ],
        out_specs=(pl.BlockSpec((C, H), lambda i: (0, 0)),
                   pl.BlockSpec((C, _DP), lambda i: (0, 0))),
        out_shape=(jax.ShapeDtypeStruct((C, H), jnp.bfloat16),
                   jax.ShapeDtypeStruct((C, _DP), jnp.int32)),
    )(class_anchors, W_proj)

    idx0 = _sim_half(features, anorm, 0)
    idx1 = _sim_half(features, anorm, _GB)

    g0 = _gather_call()(ap, idx0.reshape(_NW * _NCH, _CH))
    g1 = _gather_call()(ap, idx1.reshape(_NW * _NCH, _CH))

    out_a = pl.pallas_call(
        _proj_body_a,
        grid=(_GB,),
        in_specs=[
            pl.BlockSpec((BN, H), lambda i: (i, 0)),
            pl.BlockSpec((BN, _DP), lambda i: (i, 0)),
            pl.BlockSpec((D, H), lambda i: (0, 1)),
        ],
        out_specs=pl.BlockSpec((BN, D), lambda i: (i, 0)),
        out_shape=jax.ShapeDtypeStruct((N, D), jnp.float32),
    )(features, g0, W_proj)

    out = pl.pallas_call(
        _proj_body_b,
        grid=(_GB,),
        in_specs=[
            pl.BlockSpec((8, 128), lambda i: (0, 0)),
            pl.BlockSpec((BN, H), lambda i: (i + _GB, 0)),
            pl.BlockSpec((BN, _DP), lambda i: (i, 0)),
            pl.BlockSpec((D, H), lambda i: (0, 1)),
        ],
        out_specs=pl.BlockSpec((BN, D), lambda i: (i + _GB, 0)),
        out_shape=jax.ShapeDtypeStruct((N, D), jnp.float32),
        input_output_aliases={0: 0},
    )(out_a, features, g1, W_proj)
    return out


# speedup vs baseline: 1.7787x; 1.1242x over previous
import functools

import jax
import jax.numpy as jnp
from jax import lax
from jax.experimental import pallas as pl
from jax.experimental.pallas import tpu as pltpu
from jax.experimental.pallas import tpu_sc as plsc

N, H, C, D = 16384, 512, 1000, 512
BN = 2048
_EPS = 1e-8

_NC, _NS = 2, 16
_NW = _NC * _NS
_NH = N // 2
_BPW = _NH // _NW
_CH = 128
_NCH = _BPW // _CH
_DP = D // 2
_GB = _NH // BN


def _prep_body(anchors_ref, w1_ref, anorm_ref, ap_ref):
    a = anchors_ref[...]
    norm = jnp.sqrt(jnp.sum(a * a, axis=1, keepdims=True))
    anorm_ref[...] = (a / jnp.maximum(norm, _EPS)).astype(jnp.bfloat16)
    ap = lax.dot_general(
        a, w1_ref[...], (((1,), (1,)), ((), ())),
        preferred_element_type=jnp.float32)
    bits = lax.bitcast_convert_type(ap, jnp.uint32) + jnp.uint32(0x8000)
    lo = bits[:, :_DP] >> 16
    hi = bits[:, _DP:] & jnp.uint32(0xFFFF0000)
    ap_ref[...] = lax.bitcast_convert_type(lo | hi, jnp.int32)


def _sim_body(f_ref, anorm_ref, idx_ref):
    an = anorm_ref[...]
    hb = BN // 2
    fa = f_ref[:hb, :].astype(jnp.bfloat16)
    fb = f_ref[hb:, :].astype(jnp.bfloat16)
    dn = (((1,), (1,)), ((), ()))
    sim_a = lax.dot_general(fa, an, dn, preferred_element_type=jnp.float32)
    sim_b = lax.dot_general(fb, an, dn, preferred_element_type=jnp.float32)
    ia = jnp.argmax(sim_a, axis=-1).astype(jnp.int32)
    ib = jnp.argmax(sim_b, axis=-1).astype(jnp.int32)
    idx_ref[...] = jnp.concatenate([ia, ib])[None, None, :]


def _unpack_add(f_ref, g_ref, w2_ref, out_ref):
    gw = g_ref[...]
    lo = lax.bitcast_convert_type(gw << 16, jnp.float32)
    hi = lax.bitcast_convert_type(gw & jnp.int32(-65536), jnp.float32)
    g = jnp.concatenate([lo, hi], axis=1)
    out_ref[...] = g + lax.dot_general(
        f_ref[...], w2_ref[...], (((1,), (1,)), ((), ())),
        preferred_element_type=jnp.float32)


def _proj_body_a(f_ref, g_ref, w2_ref, out_ref):
    _unpack_add(f_ref, g_ref, w2_ref, out_ref)


def _proj_body_b(prev_ref, f_ref, g_ref, w2_ref, out_ref):
    del prev_ref
    _unpack_add(f_ref, g_ref, w2_ref, out_ref)


def _gather_body(ap_hbm, idx_hbm, out_hbm, idx_v, rows_v, sem0, sem1):
    wid = lax.axis_index("s") * _NC + lax.axis_index("c")
    pltpu.sync_copy(idx_hbm.at[pl.ds(wid * _NCH, _NCH)], idx_v)
    base = wid * _BPW
    sems = (sem0, sem1)
    handles = [None] * _NCH
    handles[0] = pltpu.async_copy(ap_hbm.at[idx_v.at[0]], rows_v.at[0], sem0)
    for j in range(_NCH):
        if j + 1 < _NCH:
            handles[j + 1] = pltpu.async_copy(
                ap_hbm.at[idx_v.at[j + 1]], rows_v.at[(j + 1) % 2],
                sems[(j + 1) % 2])
        handles[j].wait()
        pltpu.sync_copy(rows_v.at[j % 2],
                        out_hbm.at[pl.ds(base + j * _CH, _CH)])


@functools.cache
def _gather_call():
    return functools.partial(
        pl.kernel,
        mesh=plsc.VectorSubcoreMesh(
            core_axis_name="c", subcore_axis_name="s", num_cores=_NC),
        out_type=jax.ShapeDtypeStruct((_NH, _DP), jnp.int32),
        scratch_types=[
            pltpu.VMEM((_NCH, _CH), jnp.int32),
            pltpu.VMEM((2, _CH, _DP), jnp.int32),
            pltpu.SemaphoreType.DMA,
            pltpu.SemaphoreType.DMA,
        ],
    )(_gather_body)


def _sim_half(features, anorm, block0):
    return pl.pallas_call(
        _sim_body,
        grid=(_GB,),
        in_specs=[
            pl.BlockSpec((BN, H), lambda i: (i + block0, 0)),
            pl.BlockSpec((C, H), lambda i: (0, 0)),
        ],
        out_specs=pl.BlockSpec((1, 1, BN), lambda i: (i, 0, 0)),
        out_shape=jax.ShapeDtypeStruct((_GB, 1, BN), jnp.int32),
    )(features, anorm)


def kernel(features, class_anchors, W_proj):
    anorm, ap = pl.pallas_call(
        _prep_body,
        grid=(1,),
        in_specs=[
            pl.BlockSpec((C, H), lambda i: (0, 0)),
            pl.BlockSpec((D, H), lambda i: (0, 0)),
        ],
        out_specs=(pl.BlockSpec((C, H), lambda i: (0, 0)),
                   pl.BlockSpec((C, _DP), lambda i: (0, 0))),
        out_shape=(jax.ShapeDtypeStruct((C, H), jnp.bfloat16),
                   jax.ShapeDtypeStruct((C, _DP), jnp.int32)),
    )(class_anchors, W_proj)

    idx0 = _sim_half(features, anorm, 0)
    idx1 = _sim_half(features, anorm, _GB)

    g0 = _gather_call()(ap, idx0.reshape(_NW * _NCH, _CH))
    g1 = _gather_call()(ap, idx1.reshape(_NW * _NCH, _CH))

    out_a = pl.pallas_call(
        _proj_body_a,
        grid=(_GB,),
        in_specs=[
            pl.BlockSpec((BN, H), lambda i: (i, 0)),
            pl.BlockSpec((BN, _DP), lambda i: (i, 0)),
            pl.BlockSpec((D, H), lambda i: (0, 1)),
        ],
        out_specs=pl.BlockSpec((BN, D), lambda i: (i, 0)),
        out_shape=jax.ShapeDtypeStruct((N, D), jnp.float32),
    )(features, g0, W_proj)

    out = pl.pallas_call(
        _proj_body_b,
        grid=(_GB,),
        in_specs=[
            pl.BlockSpec((8, 128), lambda i: (0, 0)),
            pl.BlockSpec((BN, H), lambda i: (i + _GB, 0)),
            pl.BlockSpec((BN, _DP), lambda i: (i, 0)),
            pl.BlockSpec((D, H), lambda i: (0, 1)),
        ],
        out_specs=pl.BlockSpec((BN, D), lambda i: (i + _GB, 0)),
        out_shape=jax.ShapeDtypeStruct((N, D), jnp.float32),
        input_output_aliases={0: 0},
    )(out_a, features, g1, W_proj)
    return out


# speedup vs baseline: 1.8390x; 1.0339x over previous
import functools

import jax
import jax.numpy as jnp
from jax import lax
from jax.experimental import pallas as pl
from jax.experimental.pallas import tpu as pltpu
from jax.experimental.pallas import tpu_sc as plsc

N, H, C, D = 16384, 512, 1000, 512
BN = 2048
_EPS = 1e-8

_NC, _NS = 2, 16
_NW = _NC * _NS
_NH = N // 2
_BPW = _NH // _NW
_CH = 128
_NCH = _BPW // _CH
_DP = D // 2
_GB = _NH // BN


def _prep_body(anchors_ref, w1_ref, anorm_ref, ap_ref):
    a = anchors_ref[...]
    norm = jnp.sqrt(jnp.sum(a * a, axis=1, keepdims=True))
    anorm_ref[...] = (a / jnp.maximum(norm, _EPS)).astype(jnp.bfloat16)
    ap = lax.dot_general(
        a, w1_ref[...], (((1,), (1,)), ((), ())),
        preferred_element_type=jnp.float32)
    bits = lax.bitcast_convert_type(ap, jnp.uint32) + jnp.uint32(0x8000)
    lo = bits[:, :_DP] >> 16
    hi = bits[:, _DP:] & jnp.uint32(0xFFFF0000)
    ap_ref[...] = lax.bitcast_convert_type(lo | hi, jnp.int32)


def _sim_body(f_ref, anorm_ref, idx_ref):
    an = anorm_ref[...]
    qb = BN // 4
    dn = (((1,), (1,)), ((), ()))
    sims = [
        lax.dot_general(
            f_ref[k * qb:(k + 1) * qb, :].astype(jnp.bfloat16), an, dn,
            preferred_element_type=jnp.float32)
        for k in range(4)
    ]
    parts = [jnp.argmax(s, axis=-1).astype(jnp.int32) for s in sims]
    idx_ref[...] = jnp.concatenate(parts)[None, None, :]


def _unpack_add(f_ref, g_ref, w2_ref, out_ref):
    gw = g_ref[...]
    lo = lax.bitcast_convert_type(gw << 16, jnp.float32)
    hi = lax.bitcast_convert_type(gw & jnp.int32(-65536), jnp.float32)
    g = jnp.concatenate([lo, hi], axis=1)
    out_ref[...] = g + lax.dot_general(
        f_ref[...], w2_ref[...], (((1,), (1,)), ((), ())),
        preferred_element_type=jnp.float32)


def _proj_body_a(f_ref, g_ref, w2_ref, out_ref):
    _unpack_add(f_ref, g_ref, w2_ref, out_ref)


def _proj_body_b(prev_ref, f_ref, g_ref, w2_ref, out_ref):
    del prev_ref
    _unpack_add(f_ref, g_ref, w2_ref, out_ref)


def _gather_body(ap_hbm, idx_hbm, out_hbm, idx_v, rows_v, sem0, sem1):
    wid = lax.axis_index("s") * _NC + lax.axis_index("c")
    pltpu.sync_copy(idx_hbm.at[pl.ds(wid * _NCH, _NCH)], idx_v)
    base = wid * _BPW
    sems = (sem0, sem1)
    handles = [None] * _NCH
    handles[0] = pltpu.async_copy(ap_hbm.at[idx_v.at[0]], rows_v.at[0], sem0)
    for j in range(_NCH):
        if j + 1 < _NCH:
            handles[j + 1] = pltpu.async_copy(
                ap_hbm.at[idx_v.at[j + 1]], rows_v.at[(j + 1) % 2],
                sems[(j + 1) % 2])
        handles[j].wait()
        pltpu.sync_copy(rows_v.at[j % 2],
                        out_hbm.at[pl.ds(base + j * _CH, _CH)])


@functools.cache
def _gather_call():
    return functools.partial(
        pl.kernel,
        mesh=plsc.VectorSubcoreMesh(
            core_axis_name="c", subcore_axis_name="s", num_cores=_NC),
        out_type=jax.ShapeDtypeStruct((_NH, _DP), jnp.int32),
        scratch_types=[
            pltpu.VMEM((_NCH, _CH), jnp.int32),
            pltpu.VMEM((2, _CH, _DP), jnp.int32),
            pltpu.SemaphoreType.DMA,
            pltpu.SemaphoreType.DMA,
        ],
    )(_gather_body)


def _sim_half(features, anorm, block0):
    return pl.pallas_call(
        _sim_body,
        grid=(_GB,),
        in_specs=[
            pl.BlockSpec((BN, H), lambda i: (i + block0, 0)),
            pl.BlockSpec((C, H), lambda i: (0, 0)),
        ],
        out_specs=pl.BlockSpec((1, 1, BN), lambda i: (i, 0, 0)),
        out_shape=jax.ShapeDtypeStruct((_GB, 1, BN), jnp.int32),
    )(features, anorm)


def kernel(features, class_anchors, W_proj):
    anorm, ap = pl.pallas_call(
        _prep_body,
        grid=(1,),
        in_specs=[
            pl.BlockSpec((C, H), lambda i: (0, 0)),
            pl.BlockSpec((D, H), lambda i: (0, 0)),
        ],
        out_specs=(pl.BlockSpec((C, H), lambda i: (0, 0)),
                   pl.BlockSpec((C, _DP), lambda i: (0, 0))),
        out_shape=(jax.ShapeDtypeStruct((C, H), jnp.bfloat16),
                   jax.ShapeDtypeStruct((C, _DP), jnp.int32)),
    )(class_anchors, W_proj)

    idx0 = _sim_half(features, anorm, 0)
    idx1 = _sim_half(features, anorm, _GB)

    g0 = _gather_call()(ap, idx0.reshape(_NW * _NCH, _CH))
    g1 = _gather_call()(ap, idx1.reshape(_NW * _NCH, _CH))

    out_a = pl.pallas_call(
        _proj_body_a,
        grid=(_GB,),
        in_specs=[
            pl.BlockSpec((BN, H), lambda i: (i, 0)),
            pl.BlockSpec((BN, _DP), lambda i: (i, 0)),
            pl.BlockSpec((D, H), lambda i: (0, 1)),
        ],
        out_specs=pl.BlockSpec((BN, D), lambda i: (i, 0)),
        out_shape=jax.ShapeDtypeStruct((N, D), jnp.float32),
    )(features, g0, W_proj)

    out = pl.pallas_call(
        _proj_body_b,
        grid=(_GB,),
        in_specs=[
            pl.BlockSpec((8, 128), lambda i: (0, 0)),
            pl.BlockSpec((BN, H), lambda i: (i + _GB, 0)),
            pl.BlockSpec((BN, _DP), lambda i: (i, 0)),
            pl.BlockSpec((D, H), lambda i: (0, 1)),
        ],
        out_specs=pl.BlockSpec((BN, D), lambda i: (i + _GB, 0)),
        out_shape=jax.ShapeDtypeStruct((N, D), jnp.float32),
        input_output_aliases={0: 0},
    )(out_a, features, g1, W_proj)
    return out
